# baseline (speedup 1.0000x reference)
import functools

import jax
import jax.numpy as jnp
from jax import lax
from jax.experimental import pallas as pl
from jax.experimental.pallas import tpu as pltpu
from jax.experimental.pallas import tpu_sc as plsc

NC, NS = 2, 16
NW = NC * NS
BATCH = 128
TOTAL = 1024 * 50
NBATCH = TOTAL // BATCH
GMAX = -(-NBATCH // NW)
D = 128
DN = 32

_mesh = plsc.VectorSubcoreMesh(
    core_axis_name="c", subcore_axis_name="s", num_cores=NC, num_subcores=NS)


@functools.partial(
    pl.kernel,
    out_type=jax.ShapeDtypeStruct((TOTAL, D), jnp.float32),
    mesh=_mesh,
    scratch_types=[
        [pltpu.VMEM((BATCH,), jnp.int32)] * 3,
        [pltpu.VMEM((BATCH, D), jnp.float32)] * 3,
        [pltpu.VMEM((BATCH, DN), jnp.float32)] * 3,
        [pltpu.SemaphoreType.DMA] * 3,
        [pltpu.SemaphoreType.DMA] * 3,
        [pltpu.SemaphoreType.DMA] * 3,
        [pltpu.SemaphoreType.DMA] * 3,
    ],
    compiler_params=pltpu.CompilerParams(use_tc_tiling_on_sc=False),
)
def _fused_lookup(ids_hbm, text_hbm, id_hbm, out_hbm,
                  idx_v, trow_v, irow_v, sem_t, sem_i, sem_o, sem_x):
    wid = lax.axis_index("s") * NC + lax.axis_index("c")

    def guarded(g, fn):
        if (g + 1) * NW <= NBATCH:
            fn()
        else:
            pl.when(g * NW + wid < NBATCH)(fn)

    def start_idx(g, b):
        def _():
            base = (g * NW + wid) * BATCH
            pltpu.async_copy(ids_hbm.at[pl.ds(base, BATCH)], idx_v[b], sem_x[b])
        guarded(g, _)

    def start_gathers(g, b):
        def _():
            base = (g * NW + wid) * BATCH
            pltpu.make_async_copy(
                ids_hbm.at[pl.ds(base, BATCH)], idx_v[b], sem_x[b]).wait()
            pltpu.async_copy(text_hbm.at[idx_v[b]], trow_v[b], sem_t[b])
            pltpu.async_copy(id_hbm.at[idx_v[b]], irow_v[b], sem_i[b])
        guarded(g, _)

    def wait_fetch(g, b):
        def _():
            pltpu.make_async_copy(text_hbm.at[idx_v[b]], trow_v[b], sem_t[b]).wait()
            pltpu.make_async_copy(id_hbm.at[idx_v[b]], irow_v[b], sem_i[b]).wait()
        guarded(g, _)

    def out_slice(g):
        return out_hbm.at[pl.ds((g * NW + wid) * BATCH, BATCH)]

    def wait_scatter(g, b):
        guarded(g, lambda: pltpu.make_async_copy(trow_v[b], out_slice(g), sem_o[b]).wait())

    start_idx(0, 0)
    start_idx(1, 1)
    start_gathers(0, 0)
    for g in range(GMAX):
        b = g % 3
        wait_fetch(g, b)

        def compute(b=b):
            @plsc.parallel_loop(0, BATCH, 1, unroll=8)
            def _row(r):
                trow_v[b][r, pl.ds(D - DN, 16)] += irow_v[b][r, pl.ds(0, 16)]
                trow_v[b][r, pl.ds(D - 16, 16)] += irow_v[b][r, pl.ds(16, 16)]
            pltpu.async_copy(trow_v[b], out_slice(g), sem_o[b])
        guarded(g, functools.partial(compute))

        if g >= 1 and g + 1 < GMAX:
            wait_scatter(g - 1, (g - 1) % 3)
        if g + 2 < GMAX:
            start_idx(g + 2, (g + 2) % 3)
        if g + 1 < GMAX:
            start_gathers(g + 1, (g + 1) % 3)

    for g in range(max(GMAX - 2, 0), GMAX):
        wait_scatter(g, g % 3)


ITEMS = 100001
_TC = 512
_CG = _TC // 16
_NCHUNK = -(-ITEMS // _TC)
_TGMAX = -(-_NCHUNK // NW)


@functools.partial(
    pl.kernel,
    out_type=jax.ShapeDtypeStruct((_NCHUNK * _TC * DN,), jnp.float32),
    mesh=_mesh,
    scratch_types=[
        pltpu.VMEM((DN * _TC,), jnp.float32),
        pltpu.VMEM((_TC * DN,), jnp.float32),
        pltpu.SemaphoreType.DMA,
    ],
    compiler_params=pltpu.CompilerParams(
        use_tc_tiling_on_sc=True, disable_bounds_checks=True,
        needs_layout_passes=False),
)
def _transpose_id(idt_hbm, out_hbm, buf_v, obuf_v, sem):
    wid = lax.axis_index("s") * NC + lax.axis_index("c")
    lane = lax.broadcasted_iota(jnp.int32, (16,), 0)
    lane_dn = lane * DN
    for g in range(_TGMAX):
        cid = g * NW + wid

        def _(cid=cid):
            c0 = cid * _TC
            copies = [
                pltpu.async_copy(idt_hbm.at[k, pl.ds(c0, _TC)],
                                 buf_v.at[pl.ds(k * _TC, _TC)], sem)
                for k in range(DN)
            ]
            for c in copies:
                c.wait()

            @plsc.parallel_loop(0, _CG, 1, unroll=2)
            def _t(cg):
                base = cg * (16 * DN)
                for k in range(DN):
                    v = buf_v[pl.ds(k * _TC + cg * 16, 16)]
                    plsc.store_scatter(obuf_v, [lane_dn + (base + k)], v)

            pltpu.sync_copy(obuf_v, out_hbm.at[pl.ds(c0 * DN, _TC * DN)])

        if (g + 1) * NW <= _NCHUNK:
            _()
        else:
            pl.when(cid < _NCHUNK)(_)


def kernel(id, text_table, id_table):
    ids_t = id.T.reshape(-1)
    out = _fused_lookup(ids_t, text_table, id_table)
    return out.reshape(id.shape[1], id.shape[0], D).transpose(1, 0, 2)


# speedup vs baseline: 1.0864x; 1.0864x over previous
import functools

import jax
import jax.numpy as jnp
from jax import lax
from jax.experimental import pallas as pl
from jax.experimental.pallas import tpu as pltpu
from jax.experimental.pallas import tpu_sc as plsc

NC, NS = 2, 16
NW = NC * NS
BATCH = 128
TOTAL = 1024 * 50
NBATCH = TOTAL // BATCH
GMAX = -(-NBATCH // NW)
D = 128
DN = 32

_mesh = plsc.VectorSubcoreMesh(
    core_axis_name="c", subcore_axis_name="s", num_cores=NC, num_subcores=NS)


@functools.partial(
    pl.kernel,
    out_type=jax.ShapeDtypeStruct((TOTAL, D), jnp.float32),
    mesh=_mesh,
    scratch_types=[
        [pltpu.VMEM((BATCH,), jnp.int32)] * 3,
        [pltpu.VMEM((BATCH, D), jnp.float32)] * 3,
        [pltpu.VMEM((BATCH, DN), jnp.float32)] * 3,
        [pltpu.SemaphoreType.DMA] * 3,
        [pltpu.SemaphoreType.DMA] * 3,
        [pltpu.SemaphoreType.DMA] * 3,
    ],
    compiler_params=pltpu.CompilerParams(use_tc_tiling_on_sc=False),
)
def _fused_lookup(ids_hbm, text_hbm, id_hbm, out_hbm,
                  idx_v, trow_v, irow_v, sem_t, sem_i, sem_o):
    wid = lax.axis_index("s") * NC + lax.axis_index("c")

    def guarded(g, fn):
        if (g + 1) * NW <= NBATCH:
            fn()
        else:
            pl.when(g * NW + wid < NBATCH)(fn)

    def start_fetch(g, b):
        def _():
            base = (g * NW + wid) * BATCH
            pltpu.sync_copy(ids_hbm.at[pl.ds(base, BATCH)], idx_v[b])
            pltpu.async_copy(text_hbm.at[idx_v[b]], trow_v[b], sem_t[b])
            pltpu.async_copy(id_hbm.at[idx_v[b]], irow_v[b], sem_i[b])
        guarded(g, _)

    def wait_fetch(g, b):
        def _():
            pltpu.make_async_copy(text_hbm.at[idx_v[b]], trow_v[b], sem_t[b]).wait()
            pltpu.make_async_copy(id_hbm.at[idx_v[b]], irow_v[b], sem_i[b]).wait()
        guarded(g, _)

    def out_slice(g):
        return out_hbm.at[pl.ds((g * NW + wid) * BATCH, BATCH)]

    def wait_scatter(g, b):
        guarded(g, lambda: pltpu.make_async_copy(trow_v[b], out_slice(g), sem_o[b]).wait())

    start_fetch(0, 0)
    start_fetch(1, 1)
    for g in range(GMAX):
        b = g % 3
        wait_fetch(g, b)

        def compute(b=b):
            @plsc.parallel_loop(0, BATCH, 1, unroll=8)
            def _row(r):
                trow_v[b][r, pl.ds(D - DN, 16)] += irow_v[b][r, pl.ds(0, 16)]
                trow_v[b][r, pl.ds(D - 16, 16)] += irow_v[b][r, pl.ds(16, 16)]
            pltpu.async_copy(trow_v[b], out_slice(g), sem_o[b])
        guarded(g, functools.partial(compute))

        if g + 2 < GMAX:
            if g >= 1:
                wait_scatter(g - 1, (g - 1) % 3)
            start_fetch(g + 2, (g + 2) % 3)

    for g in range(max(GMAX - 3, 0), GMAX):
        wait_scatter(g, g % 3)


def kernel(id, text_table, id_table):
    ids_t = id.T.reshape(-1)
    out = _fused_lookup(ids_t, text_table, id_table)
    return out.reshape(id.shape[1], id.shape[0], D).transpose(1, 0, 2)


# speedup vs baseline: 1.1042x; 1.0163x over previous
import functools

import jax
import jax.numpy as jnp
from jax import lax
from jax.experimental import pallas as pl
from jax.experimental.pallas import tpu as pltpu
from jax.experimental.pallas import tpu_sc as plsc

NC, NS = 2, 16
NW = NC * NS
BATCH = 128
TOTAL = 1024 * 50
NBATCH = TOTAL // BATCH
GMAX = -(-NBATCH // NW)
D = 128
DN = 32

_mesh = plsc.VectorSubcoreMesh(
    core_axis_name="c", subcore_axis_name="s", num_cores=NC, num_subcores=NS)


@functools.partial(
    pl.kernel,
    out_type=jax.ShapeDtypeStruct((TOTAL, D), jnp.float32),
    mesh=_mesh,
    scratch_types=[
        [pltpu.VMEM((BATCH,), jnp.int32)] * 4,
        [pltpu.VMEM((BATCH, D), jnp.float32)] * 4,
        [pltpu.VMEM((BATCH, DN), jnp.float32)] * 4,
        [pltpu.SemaphoreType.DMA] * 4,
        [pltpu.SemaphoreType.DMA] * 4,
        [pltpu.SemaphoreType.DMA] * 4,
    ],
    compiler_params=pltpu.CompilerParams(use_tc_tiling_on_sc=False),
)
def _fused_lookup(ids_hbm, text_hbm, id_hbm, out_hbm,
                  idx_v, trow_v, irow_v, sem_t, sem_i, sem_o):
    wid = lax.axis_index("s") * NC + lax.axis_index("c")

    def guarded(g, fn):
        if (g + 1) * NW <= NBATCH:
            fn()
        else:
            pl.when(g * NW + wid < NBATCH)(fn)

    def start_fetch(g, b):
        def _():
            base = (g * NW + wid) * BATCH
            pltpu.sync_copy(ids_hbm.at[pl.ds(base, BATCH)], idx_v[b])
            pltpu.async_copy(text_hbm.at[idx_v[b]], trow_v[b], sem_t[b])
            pltpu.async_copy(id_hbm.at[idx_v[b]], irow_v[b], sem_i[b])
        guarded(g, _)

    def wait_fetch(g, b):
        def _():
            pltpu.make_async_copy(text_hbm.at[idx_v[b]], trow_v[b], sem_t[b]).wait()
            pltpu.make_async_copy(id_hbm.at[idx_v[b]], irow_v[b], sem_i[b]).wait()
        guarded(g, _)

    def out_slice(g):
        return out_hbm.at[pl.ds((g * NW + wid) * BATCH, BATCH)]

    def wait_scatter(g, b):
        guarded(g, lambda: pltpu.make_async_copy(trow_v[b], out_slice(g), sem_o[b]).wait())

    start_fetch(0, 0)
    start_fetch(1, 1)
    start_fetch(2, 2)
    for g in range(GMAX):
        b = g % 4
        wait_fetch(g, b)

        def compute(b=b):
            @plsc.parallel_loop(0, BATCH, 1, unroll=8)
            def _row(r):
                trow_v[b][r, pl.ds(D - DN, 16)] += irow_v[b][r, pl.ds(0, 16)]
                trow_v[b][r, pl.ds(D - 16, 16)] += irow_v[b][r, pl.ds(16, 16)]
            pltpu.async_copy(trow_v[b], out_slice(g), sem_o[b])
        guarded(g, functools.partial(compute))

        if g + 3 < GMAX:
            if g >= 1:
                wait_scatter(g - 1, (g - 1) % 4)
            start_fetch(g + 3, (g + 3) % 4)

    for g in range(max(GMAX - 4, 0), GMAX):
        wait_scatter(g, g % 4)


def kernel(id, text_table, id_table):
    ids_t = id.T.reshape(-1)
    out = _fused_lookup(ids_t, text_table, id_table)
    return out.reshape(id.shape[1], id.shape[0], D).transpose(1, 0, 2)
